# baseline (speedup 1.0000x reference)
import functools

import jax
import jax.numpy as jnp
from jax import lax
from jax.experimental import pallas as pl
from jax.experimental.pallas import tpu as pltpu
from jax.experimental.pallas import tpu_sc as plsc

_N = 10000
_E = 320000
_REFRACTORY = 0.33
_RELU_CLIP = 1.0
_NW = 32
_EPW = _E // _NW
_L = 16
_G = 32
_EPW_PAD = -(-_EPW // 256) * 256
_GRP = _EPW_PAD // _G


def _sc_edge_kernel(state_hbm, w_hbm, src_hbm, dst_hbm, oid_hbm, bias_hbm,
                    act_hbm, part_out, base_out, actv_out,
                    state_v, src_v, dst_v, w_v, row_v, oid_v, bias_v,
                    act_v, base_v, acti_v, sem, sem2):
    wid = lax.axis_index("s") * 2 + lax.axis_index("c")
    c_state = pltpu.async_copy(state_hbm, state_v, sem)
    c_src = pltpu.async_copy(
        src_hbm.at[pl.ds(wid * _EPW_PAD, _EPW_PAD)], src_v, sem)
    c_dst = pltpu.async_copy(
        dst_hbm.at[pl.ds(wid * _EPW_PAD, _EPW_PAD)], dst_v, sem)
    c_w = pltpu.async_copy(
        w_hbm.at[pl.ds(wid * _EPW, _EPW)], w_v.at[pl.ds(0, _EPW)], sem)
    c_oid = pltpu.async_copy(oid_hbm, oid_v, sem)
    c_oid.wait()

    @pl.when(wid == 0)
    def _():
        pltpu.async_copy(bias_hbm, bias_v, sem2)
        pltpu.async_copy(act_hbm, act_v, sem2)

    lane = lax.iota(jnp.int32, _L)
    o0 = plsc.load_gather(oid_v, [jnp.full((_L,), 0, jnp.int32)])
    o1 = plsc.load_gather(oid_v, [jnp.full((_L,), 1, jnp.int32)])
    o2 = plsc.load_gather(oid_v, [jnp.full((_L,), 2, jnp.int32)])
    o3 = plsc.load_gather(oid_v, [jnp.full((_L,), 3, jnp.int32)])
    fmt = plsc.PackFormat.INTERLEAVED
    zero = jnp.zeros((_L,), jnp.float32)

    c_state.wait()
    c_src.wait()
    c_dst.wait()
    c_w.wait()

    def body(g, carry):
        d16 = dst_v[pl.ds(g * _G, _G)]
        d_lo, d_hi = plsc.unpack(d16, format=fmt)
        h0l, h0h = d_lo == o0, d_hi == o0
        h1l, h1h = d_lo == o1, d_hi == o1
        h2l, h2h = d_lo == o2, d_hi == o2
        h3l, h3h = d_lo == o3, d_hi == o3
        hit = ((h0l | h0h) | (h1l | h1h)) | ((h2l | h2h) | (h3l | h3h))

        def slow(args):
            a0, a1, a2, a3 = args
            s_lo, s_hi = plsc.unpack(src_v[pl.ds(g * _G, _G)], format=fmt)
            w_lo = w_v[pl.ds(g * _G, _L)]
            w_hi = w_v[pl.ds(g * _G + _L, _L)]
            m_lo = w_lo * plsc.load_gather(state_v, [s_lo])
            m_hi = w_hi * plsc.load_gather(state_v, [s_hi])
            a0 = a0 + jnp.where(h0l, m_lo, zero) + jnp.where(h0h, m_hi, zero)
            a1 = a1 + jnp.where(h1l, m_lo, zero) + jnp.where(h1h, m_hi, zero)
            a2 = a2 + jnp.where(h2l, m_lo, zero) + jnp.where(h2h, m_hi, zero)
            a3 = a3 + jnp.where(h3l, m_lo, zero) + jnp.where(h3h, m_hi, zero)
            return a0, a1, a2, a3

        return lax.cond(jnp.any(hit), slow, lambda args: args, carry)

    a0, a1, a2, a3 = lax.fori_loop(0, _GRP, body, (zero, zero, zero, zero))
    t0, t1, t2, t3 = jnp.sum(a0), jnp.sum(a1), jnp.sum(a2), jnp.sum(a3)
    row = jnp.where(lane == 0, t0,
          jnp.where(lane == 1, t1,
          jnp.where(lane == 2, t2,
          jnp.where(lane == 3, t3, 0.0)))) * _REFRACTORY
    row_v[...] = row
    pltpu.sync_copy(row_v, part_out.at[wid])

    @pl.when(wid == 0)
    def _():
        pltpu.make_async_copy(bias_hbm, bias_v, sem2).wait()
        pltpu.make_async_copy(act_hbm, act_v, sem2).wait()
        oid_vec = plsc.load_gather(oid_v, [jnp.minimum(lane, 3)])
        pv = plsc.load_gather(state_v, [oid_vec]) * _REFRACTORY
        bv = plsc.load_gather(bias_v, [oid_vec])
        base_v[...] = pv + bv
        acti_v[...] = plsc.load_gather(act_v, [oid_vec])
        pltpu.sync_copy(base_v, base_out.at[0])
        pltpu.sync_copy(acti_v, actv_out.at[0])


_sc_edge_call = functools.partial(
    pl.kernel,
    mesh=plsc.VectorSubcoreMesh(core_axis_name="c", subcore_axis_name="s"),
    compiler_params=pltpu.CompilerParams(needs_layout_passes=False),
    out_type=[
        jax.ShapeDtypeStruct((_NW, _L), jnp.float32),
        jax.ShapeDtypeStruct((1, _L), jnp.float32),
        jax.ShapeDtypeStruct((1, _L), jnp.int32),
    ],
    scratch_types=[
        pltpu.VMEM((_N,), jnp.float32),
        pltpu.VMEM((_EPW_PAD,), jnp.int16),
        pltpu.VMEM((_EPW_PAD,), jnp.int16),
        pltpu.VMEM((_EPW_PAD,), jnp.float32),
        pltpu.VMEM((_L,), jnp.float32),
        pltpu.VMEM((_L,), jnp.int32),
        pltpu.VMEM((_N,), jnp.float32),
        pltpu.VMEM((_N,), jnp.int32),
        pltpu.VMEM((_L,), jnp.float32),
        pltpu.VMEM((_L,), jnp.int32),
        pltpu.SemaphoreType.DMA,
        pltpu.SemaphoreType.DMA,
    ],
)(_sc_edge_kernel)


def _interleave16(x, pad_val):
    x2 = x.reshape(_NW, _EPW).astype(jnp.int16)
    x2 = jnp.pad(x2, ((0, 0), (0, _EPW_PAD - _EPW)), constant_values=pad_val)
    x2 = x2.reshape(_NW, _GRP, 2, _L).transpose(0, 1, 3, 2)
    return x2.reshape(-1)


def _tc_finish_kernel(part_ref, base_ref, act_ref, out_ref):
    x = jnp.sum(part_ref[...], axis=0, keepdims=True) + base_ref[...]
    a = act_ref[...]
    r = x
    r = jnp.where(a == 1, jnp.maximum(x, 0.0), r)
    r = jnp.where(a == 2, jnp.where(x >= 0, x, 0.01 * x), r)
    r = jnp.where(a == 3, jnp.clip(x, 0.0, _RELU_CLIP), r)
    r = jnp.where(a == 4, jnp.tanh(x), r)
    r = jnp.where(a == 5, jax.nn.sigmoid(x), r)
    r = jnp.where(a == 6, jnp.maximum(x, 0.0) + jnp.log1p(jnp.exp(-jnp.abs(x))), r)
    r = jnp.where(a == 7, jnp.abs(x), r)
    out_ref[...] = r


def kernel(input, state, weights, biases, src, dst, act_ids, output_ids):
    del input
    src16 = _interleave16(src.astype(jnp.int32), 0)
    dst16 = _interleave16(dst.astype(jnp.int32), -1)
    oid16 = jnp.concatenate(
        [output_ids.astype(jnp.int32),
         jnp.zeros((_L - output_ids.shape[0],), jnp.int32)])
    part, base, actv = _sc_edge_call(
        state, weights, src16, dst16, oid16, biases, act_ids)
    res = pl.pallas_call(
        _tc_finish_kernel,
        out_shape=jax.ShapeDtypeStruct((1, _L), jnp.float32),
    )(part, base, actv)
    return res[0, :4]


# speedup vs baseline: 8.2283x; 8.2283x over previous
import functools

import jax
import jax.numpy as jnp
from jax import lax
from jax.experimental import pallas as pl
from jax.experimental.pallas import tpu as pltpu
from jax.experimental.pallas import tpu_sc as plsc

_N = 10000
_E = 320000
_REFRACTORY = 0.33
_RELU_CLIP = 1.0
_NW = 32
_EPW = _E // _NW
_L = 16
_G = 2 * _L
_GRP = _EPW // _G


def _sc_edge_kernel(state_hbm, w_hbm, src_hbm, dst_hbm, oid_hbm, bias_hbm,
                    act_hbm, part_out, base_out, actv_out,
                    state_v, src_v, dst_v, w_v, row_v, oid_v, bias_v,
                    act_v, base_v, acti_v, sem, sem2):
    wid = lax.axis_index("s") * 2 + lax.axis_index("c")
    c_state = pltpu.async_copy(state_hbm, state_v, sem)
    c_src = pltpu.async_copy(src_hbm.at[pl.ds(wid * _EPW, _EPW)], src_v, sem)
    c_dst = pltpu.async_copy(dst_hbm.at[pl.ds(wid * _EPW, _EPW)], dst_v, sem)
    c_w = pltpu.async_copy(w_hbm.at[pl.ds(wid * _EPW, _EPW)], w_v, sem)
    c_oid = pltpu.async_copy(oid_hbm, oid_v, sem2)

    @pl.when(wid == 0)
    def _():
        pltpu.async_copy(bias_hbm, bias_v, sem2)
        pltpu.async_copy(act_hbm, act_v, sem2)

    c_oid.wait()
    lane = lax.iota(jnp.int32, _L)
    o0 = plsc.load_gather(oid_v, [jnp.full((_L,), 0, jnp.int32)])
    o1 = plsc.load_gather(oid_v, [jnp.full((_L,), 1, jnp.int32)])
    o2 = plsc.load_gather(oid_v, [jnp.full((_L,), 2, jnp.int32)])
    o3 = plsc.load_gather(oid_v, [jnp.full((_L,), 3, jnp.int32)])
    zero = jnp.zeros((_L,), jnp.float32)

    c_state.wait()
    c_src.wait()
    c_dst.wait()
    c_w.wait()

    def accum(args, b, d, h0, h1, h2, h3):
        a0, a1, a2, a3 = args
        s = src_v[pl.ds(b, _L)]
        w = w_v[pl.ds(b, _L)]
        m = w * plsc.load_gather(state_v, [s])
        return (a0 + jnp.where(h0, m, zero), a1 + jnp.where(h1, m, zero),
                a2 + jnp.where(h2, m, zero), a3 + jnp.where(h3, m, zero))

    def body(g, carry):
        b = g * _G
        d0 = dst_v[pl.ds(b, _L)]
        d1 = dst_v[pl.ds(b + _L, _L)]
        h00, h01, h02, h03 = d0 == o0, d0 == o1, d0 == o2, d0 == o3
        h10, h11, h12, h13 = d1 == o0, d1 == o1, d1 == o2, d1 == o3
        hit = ((h00 | h01) | (h02 | h03)) | ((h10 | h11) | (h12 | h13))

        def slow(args):
            args = accum(args, b, d0, h00, h01, h02, h03)
            return accum(args, b + _L, d1, h10, h11, h12, h13)

        return lax.cond(jnp.any(hit), slow, lambda args: args, carry)

    acc = lax.fori_loop(0, _GRP, body, (zero, zero, zero, zero))
    bt = _GRP * _G
    dt = dst_v[pl.ds(bt, _L)]
    a0, a1, a2, a3 = accum(acc, bt, dt,
                           dt == o0, dt == o1, dt == o2, dt == o3)

    t0, t1, t2, t3 = jnp.sum(a0), jnp.sum(a1), jnp.sum(a2), jnp.sum(a3)
    row = jnp.where(lane == 0, t0,
          jnp.where(lane == 1, t1,
          jnp.where(lane == 2, t2,
          jnp.where(lane == 3, t3, 0.0)))) * _REFRACTORY
    row_v[...] = row
    pltpu.sync_copy(row_v, part_out.at[wid])

    @pl.when(wid == 0)
    def _():
        pltpu.make_async_copy(bias_hbm, bias_v, sem2).wait()
        pltpu.make_async_copy(act_hbm, act_v, sem2).wait()
        oid_vec = plsc.load_gather(oid_v, [jnp.minimum(lane, 3)])
        pv = plsc.load_gather(state_v, [oid_vec]) * _REFRACTORY
        bv = plsc.load_gather(bias_v, [oid_vec])
        base_v[...] = pv + bv
        acti_v[...] = plsc.load_gather(act_v, [oid_vec])
        pltpu.sync_copy(base_v, base_out.at[0])
        pltpu.sync_copy(acti_v, actv_out.at[0])


_sc_edge_call = functools.partial(
    pl.kernel,
    mesh=plsc.VectorSubcoreMesh(core_axis_name="c", subcore_axis_name="s"),
    compiler_params=pltpu.CompilerParams(needs_layout_passes=False),
    out_type=[
        jax.ShapeDtypeStruct((_NW, _L), jnp.float32),
        jax.ShapeDtypeStruct((1, _L), jnp.float32),
        jax.ShapeDtypeStruct((1, _L), jnp.int32),
    ],
    scratch_types=[
        pltpu.VMEM((_N,), jnp.float32),
        pltpu.VMEM((_EPW,), jnp.int32),
        pltpu.VMEM((_EPW,), jnp.int32),
        pltpu.VMEM((_EPW,), jnp.float32),
        pltpu.VMEM((_L,), jnp.float32),
        pltpu.VMEM((_L,), jnp.int32),
        pltpu.VMEM((_N,), jnp.float32),
        pltpu.VMEM((_N,), jnp.int32),
        pltpu.VMEM((_L,), jnp.float32),
        pltpu.VMEM((_L,), jnp.int32),
        pltpu.SemaphoreType.DMA,
        pltpu.SemaphoreType.DMA,
    ],
)(_sc_edge_kernel)


def _tc_finish_kernel(part_ref, base_ref, act_ref, out_ref):
    x = jnp.sum(part_ref[...], axis=0, keepdims=True) + base_ref[...]
    a = act_ref[...]
    r = x
    r = jnp.where(a == 1, jnp.maximum(x, 0.0), r)
    r = jnp.where(a == 2, jnp.where(x >= 0, x, 0.01 * x), r)
    r = jnp.where(a == 3, jnp.clip(x, 0.0, _RELU_CLIP), r)
    r = jnp.where(a == 4, jnp.tanh(x), r)
    r = jnp.where(a == 5, jax.nn.sigmoid(x), r)
    r = jnp.where(a == 6, jnp.maximum(x, 0.0) + jnp.log1p(jnp.exp(-jnp.abs(x))), r)
    r = jnp.where(a == 7, jnp.abs(x), r)
    out_ref[...] = r


def kernel(input, state, weights, biases, src, dst, act_ids, output_ids):
    del input
    src = src.astype(jnp.int32)
    dst = dst.astype(jnp.int32)
    oid16 = jnp.concatenate(
        [output_ids.astype(jnp.int32),
         jnp.zeros((_L - output_ids.shape[0],), jnp.int32)])
    part, base, actv = _sc_edge_call(
        state, weights, src, dst, oid16, biases, act_ids)
    res = pl.pallas_call(
        _tc_finish_kernel,
        out_shape=jax.ShapeDtypeStruct((1, _L), jnp.float32),
    )(part, base, actv)
    return res[0, :4]


# speedup vs baseline: 10.0358x; 1.2197x over previous
import functools

import jax
import jax.numpy as jnp
from jax import lax
from jax.experimental import pallas as pl
from jax.experimental.pallas import tpu as pltpu
from jax.experimental.pallas import tpu_sc as plsc

_N = 10000
_E = 320000
_REFRACTORY = 0.33
_RELU_CLIP = 1.0
_NT = 16
_EPT = _E // _NT
_L = 16
_G = 2 * _L
_GRP = _EPT // _G


def _log1p_newton(z):
    y = 1.0 + z
    t = z * 0.6931472
    for _ in range(4):
        t = t - 1.0 + y * jnp.exp(-t)
    return t


def _sc_kernel(state_hbm, w_hbm, src_hbm, dst_hbm, oid_hbm, bias_hbm,
               act_hbm, out_hbm,
               state_v, src_v, dst_v, w_v, row_v, oid_v, bias_v,
               act_v, sum_v, shared, hits, sem, sem_oid, sem_ba):
    wid = lax.axis_index("s")
    c_dst = pltpu.async_copy(dst_hbm.at[pl.ds(wid * _EPT, _EPT)], dst_v, sem)
    c_src = pltpu.async_copy(src_hbm.at[pl.ds(wid * _EPT, _EPT)], src_v, sem)
    c_w = pltpu.async_copy(w_hbm.at[pl.ds(wid * _EPT, _EPT)], w_v, sem)
    c_state = pltpu.async_copy(state_hbm, state_v, sem)
    c_oid = pltpu.async_copy(oid_hbm, oid_v, sem_oid)

    @pl.when(wid == 0)
    def _():
        pltpu.async_copy(bias_hbm, bias_v, sem_ba)
        pltpu.async_copy(act_hbm, act_v, sem_ba)

    c_oid.wait()
    lane = lax.iota(jnp.int32, _L)
    o0 = plsc.load_gather(oid_v, [jnp.full((_L,), 0, jnp.int32)])
    o1 = plsc.load_gather(oid_v, [jnp.full((_L,), 1, jnp.int32)])
    o2 = plsc.load_gather(oid_v, [jnp.full((_L,), 2, jnp.int32)])
    o3 = plsc.load_gather(oid_v, [jnp.full((_L,), 3, jnp.int32)])
    zero = jnp.zeros((_L,), jnp.float32)

    c_dst.wait()

    def scan_body(g, cnt):
        b = g * _G
        d0 = dst_v[pl.ds(b, _L)]
        d1 = dst_v[pl.ds(b + _L, _L)]
        hit = (((d0 == o0) | (d0 == o1)) | ((d0 == o2) | (d0 == o3))) \
            | (((d1 == o0) | (d1 == o1)) | ((d1 == o2) | (d1 == o3)))
        anyhit = jnp.any(hit)

        @pl.when(anyhit)
        def _():
            hits[cnt] = g

        return cnt + anyhit.astype(jnp.int32)

    n_hits = lax.fori_loop(0, _GRP, scan_body, jnp.int32(0))

    c_src.wait()
    c_w.wait()
    c_state.wait()

    def accum(args, b, d):
        a0, a1, a2, a3 = args
        s = src_v[pl.ds(b, _L)]
        w = w_v[pl.ds(b, _L)]
        m = w * plsc.load_gather(state_v, [s])
        return (a0 + jnp.where(d == o0, m, zero),
                a1 + jnp.where(d == o1, m, zero),
                a2 + jnp.where(d == o2, m, zero),
                a3 + jnp.where(d == o3, m, zero))

    def hit_body(i, carry):
        b = hits[i] * _G
        carry = accum(carry, b, dst_v[pl.ds(b, _L)])
        return accum(carry, b + _L, dst_v[pl.ds(b + _L, _L)])

    a0, a1, a2, a3 = lax.fori_loop(0, n_hits, hit_body,
                                   (zero, zero, zero, zero))

    t0, t1, t2, t3 = jnp.sum(a0), jnp.sum(a1), jnp.sum(a2), jnp.sum(a3)
    row = jnp.where(lane == 0, t0,
          jnp.where(lane == 1, t1,
          jnp.where(lane == 2, t2,
          jnp.where(lane == 3, t3, 0.0)))) * _REFRACTORY
    row_v[...] = row
    pltpu.sync_copy(row_v, shared.at[wid])
    plsc.subcore_barrier()

    @pl.when(wid == 0)
    def _():
        pltpu.make_async_copy(bias_hbm, bias_v, sem_ba).wait()
        pltpu.make_async_copy(act_hbm, act_v, sem_ba).wait()
        pltpu.sync_copy(shared, sum_v)
        x = sum_v[0, :]
        for i in range(1, _NT):
            x = x + sum_v[i, :]
        oid_vec = plsc.load_gather(oid_v, [jnp.minimum(lane, 3)])
        x = x + plsc.load_gather(state_v, [oid_vec]) * _REFRACTORY
        x = x + plsc.load_gather(bias_v, [oid_vec])
        a = plsc.load_gather(act_v, [oid_vec])
        r = x
        r = jnp.where(a == 1, jnp.maximum(x, 0.0), r)
        r = jnp.where(a == 2, jnp.where(x >= 0, x, 0.01 * x), r)
        r = jnp.where(a == 3, jnp.clip(x, 0.0, _RELU_CLIP), r)
        ez = jnp.exp(-2.0 * jnp.abs(x))
        th = (1.0 - ez) / (1.0 + ez)
        r = jnp.where(a == 4, jnp.where(x >= 0, th, -th), r)
        r = jnp.where(a == 5, 1.0 / (1.0 + jnp.exp(-x)), r)
        sp = jnp.maximum(x, 0.0) + _log1p_newton(jnp.exp(-jnp.abs(x)))
        r = jnp.where(a == 6, sp, r)
        r = jnp.where(a == 7, jnp.abs(x), r)
        row_v[...] = r
        pltpu.sync_copy(row_v, out_hbm)


_sc_call = functools.partial(
    pl.kernel,
    mesh=plsc.VectorSubcoreMesh(core_axis_name="c", subcore_axis_name="s",
                                num_cores=1),
    compiler_params=pltpu.CompilerParams(needs_layout_passes=False),
    out_type=jax.ShapeDtypeStruct((_L,), jnp.float32),
    scratch_types=[
        pltpu.VMEM((_N,), jnp.float32),
        pltpu.VMEM((_EPT,), jnp.int32),
        pltpu.VMEM((_EPT,), jnp.int32),
        pltpu.VMEM((_EPT,), jnp.float32),
        pltpu.VMEM((_L,), jnp.float32),
        pltpu.VMEM((_L,), jnp.int32),
        pltpu.VMEM((_N,), jnp.float32),
        pltpu.VMEM((_N,), jnp.int32),
        pltpu.VMEM((_NT, _L), jnp.float32),
        pltpu.VMEM_SHARED((_NT, _L), jnp.float32),
        pltpu.SMEM((_GRP,), jnp.int32),
        pltpu.SemaphoreType.DMA,
        pltpu.SemaphoreType.DMA,
        pltpu.SemaphoreType.DMA,
    ],
)(_sc_kernel)


def kernel(input, state, weights, biases, src, dst, act_ids, output_ids):
    del input
    src = src.astype(jnp.int32)
    dst = dst.astype(jnp.int32)
    oid16 = jnp.concatenate(
        [output_ids.astype(jnp.int32),
         jnp.zeros((_L - output_ids.shape[0],), jnp.int32)])
    res = _sc_call(state, weights, src, dst, oid16, biases, act_ids)
    return res[:4]
